# Initial kernel scaffold; baseline (speedup 1.0000x reference)
import jax
import jax.numpy as jnp
from jax.experimental import pallas as pl


def kernel(x, edge_index, W2, b2, W1, b1, Wlin, blin):
    raise NotImplementedError("write your pallas kernel here")


# speedup vs baseline: 12.2954x; 12.2954x over previous
import jax
import jax.numpy as jnp
from jax.experimental import pallas as pl

P = 128


def _reg_gnn_body(x_ref, adj_ref, w2t_ref, b2_ref, w1at_ref, w1bt_ref,
                  b1_ref, wlin_ref, blin_ref, out_ref):
    f32 = jnp.float32
    x = x_ref[...]
    h = jnp.dot(x, w2t_ref[...], preferred_element_type=f32) + b2_ref[...]
    m = (adj_ref[...] != 0).astype(f32)
    mt = m.T
    sums = jnp.dot(mt, h, preferred_element_type=f32)
    counts = jnp.sum(mt, axis=1, keepdims=True)
    aggr = sums / jnp.maximum(counts, 1.0)
    out = (jnp.dot(h, w1at_ref[...], preferred_element_type=f32)
           + jnp.dot(aggr, w1bt_ref[...], preferred_element_type=f32)
           + b1_ref[...])
    y = jnp.maximum(out, 0.0)
    out_ref[...] = (jnp.dot(wlin_ref[...], y, preferred_element_type=f32)
                    + blin_ref[...])


def kernel(x, edge_index, W2, b2, W1, b1, Wlin, blin):
    n, c = x.shape
    f32 = jnp.float32

    def pad2(a, rows, cols):
        return jnp.pad(a, ((0, rows - a.shape[0]), (0, cols - a.shape[1])))

    x_p = pad2(x.astype(f32), P, P)
    adj_p = pad2(edge_index.astype(jnp.int32), P, P)
    w2t_p = pad2(W2.astype(f32).T, P, P)
    b2_p = pad2(b2.astype(f32)[None, :], 1, P)
    w1at_p = pad2(W1[:, :c].astype(f32).T, P, P)
    w1bt_p = pad2(W1[:, c:].astype(f32).T, P, P)
    b1_p = pad2(b1.astype(f32)[None, :], 1, P)
    wlin_p = pad2(Wlin.astype(f32), P, P)
    blin_p = pad2(blin.astype(f32)[:, None], P, 1)

    out = pl.pallas_call(
        _reg_gnn_body,
        out_shape=jax.ShapeDtypeStruct((P, P), f32),
    )(x_p, adj_p, w2t_p, b2_p, w1at_p, w1bt_p, b1_p, wlin_p, blin_p)
    return out[:n, :c]


# speedup vs baseline: 39.1167x; 3.1814x over previous
import jax
import jax.numpy as jnp
from jax import lax
from jax.experimental import pallas as pl


def _reg_gnn_body(x_ref, adj_ref, w2_ref, b2_ref, w1_ref, b1_ref,
                  wlin_ref, blin_ref, out_ref):
    f32 = jnp.float32
    c = x_ref.shape[1]

    def dot_nt(a, b):
        return lax.dot_general(a, b, (((1,), (1,)), ((), ())),
                               preferred_element_type=f32)

    def dot_tn(a, b):
        return lax.dot_general(a, b, (((0,), (0,)), ((), ())),
                               preferred_element_type=f32)

    h = dot_nt(x_ref[...], w2_ref[...]) + b2_ref[...]
    m = (adj_ref[...] != 0).astype(f32)
    sums = dot_tn(m, h)
    counts = dot_tn(m, jnp.ones((m.shape[0], 1), f32))
    aggr = sums / jnp.maximum(counts, 1.0)
    out = (dot_nt(h, w1_ref[:, :c]) + dot_nt(aggr, w1_ref[:, c:])
           + b1_ref[...])
    y = jnp.maximum(out, 0.0)
    out_ref[...] = (jnp.dot(wlin_ref[...], y, preferred_element_type=f32)
                    + blin_ref[...])


def kernel(x, edge_index, W2, b2, W1, b1, Wlin, blin):
    n, c = x.shape
    return pl.pallas_call(
        _reg_gnn_body,
        out_shape=jax.ShapeDtypeStruct((n, c), jnp.float32),
    )(x, edge_index, W2, b2.reshape(1, c), W1, b1.reshape(1, c),
      Wlin, blin.reshape(n, 1))
